# baseline (speedup 1.0000x reference)
import functools

import jax
import jax.numpy as jnp
from jax import lax
from jax.experimental import pallas as pl
from jax.experimental.pallas import tpu as pltpu
from jax.experimental.pallas import tpu_sc as plsc

_N = 10000
_E = 320000
_D = 128
_NC = 2
_NS = 16
_NW = _NC * _NS
_C = 128
_ROWS = 2560
_EPAD = _ROWS * _C
_RPW = _ROWS // _NW
_GC = 64
_GROWS = _EPAD // _GC
_GD = 5
_ICH = 16
_R0 = _GROWS // _NS
_GNCH = _R0 // _ICH
_NP = 10112
_SPT = _NP // _NS


def _mesh():
    return plsc.VectorSubcoreMesh(
        core_axis_name="c", subcore_axis_name="s",
        num_cores=_NC, num_subcores=_NS,
    )


def _segsum_body(z_hbm, srcr, dstr, zeros_hbm, out_hbm,
                 sidx, didx, buf, acc, gsem, ssem, isem):
    c = lax.axis_index("c")
    s = lax.axis_index("s")

    @pl.when(c == 0)
    def _core0():
        base = s * _R0
        pltpu.sync_copy(zeros_hbm.at[pl.ds(s * _SPT, _SPT)],
                        acc.at[pl.ds(s * _SPT, _SPT)])
        pltpu.sync_copy(srcr.at[pl.ds(base, _ICH)], sidx.at[pl.ds(0, _ICH)])
        pltpu.sync_copy(dstr.at[pl.ds(base, _ICH)], didx.at[pl.ds(0, _ICH)])
        pltpu.async_copy(srcr.at[pl.ds(base + _ICH, _ICH)],
                         sidx.at[pl.ds(_ICH, _ICH)], isem.at[1])
        pltpu.async_copy(dstr.at[pl.ds(base + _ICH, _ICH)],
                         didx.at[pl.ds(_ICH, _ICH)], isem.at[1])
        plsc.subcore_barrier()

        def body(r, carry):
            ch = r // _ICH
            rr = lax.rem(r, _ICH)
            slot = lax.rem(ch, 2)

            @pl.when(jnp.logical_and(rr == 0,
                                     jnp.logical_and(r > 0, r < _R0)))
            def _():
                for _k in range(2):
                    pltpu.make_async_copy(srcr.at[pl.ds(base, _ICH)],
                                          sidx.at[pl.ds(0, _ICH)],
                                          isem.at[slot]).wait()

            @pl.when(jnp.logical_and(rr == _GD,
                                     jnp.logical_and(ch >= 1,
                                                     ch + 1 < _GNCH)))
            def _():
                nxt = ch + 1
                nslot = lax.rem(nxt, 2)
                pltpu.async_copy(srcr.at[pl.ds(base + nxt * _ICH, _ICH)],
                                 sidx.at[pl.ds(nslot * _ICH, _ICH)],
                                 isem.at[nslot])
                pltpu.async_copy(dstr.at[pl.ds(base + nxt * _ICH, _ICH)],
                                 didx.at[pl.ds(nslot * _ICH, _ICH)],
                                 isem.at[nslot])

            pa = lax.rem(r, _GD)
            sla = buf.at[pl.ds(pa * _GC, _GC)]

            @pl.when(r >= _GD)
            def _():
                pltpu.make_async_copy(sla, acc.at[didx.at[0]],
                                      ssem.at[pa]).wait()

            @pl.when(r < _R0)
            def _():
                pltpu.async_copy(z_hbm.at[sidx.at[slot * _ICH + rr]], sla,
                                 gsem.at[pa])

            rb = r - _GD + 1
            pb = lax.rem(rb + _GD, _GD)
            slb = buf.at[pl.ds(pb * _GC, _GC)]

            @pl.when(jnp.logical_and(rb >= 0, rb < _R0))
            def _():
                chb = rb // _ICH
                rowb = lax.rem(chb, 2) * _ICH + lax.rem(rb, _ICH)
                pltpu.make_async_copy(z_hbm.at[sidx.at[0]], slb,
                                      gsem.at[pb]).wait()
                pltpu.async_copy(slb, acc.at[didx.at[rowb]], ssem.at[pb],
                                 add=True)

            return carry

        lax.fori_loop(0, _R0 + _GD - 1, body, 0, unroll=False)
        pl_last = (_R0 - 1) % _GD
        pltpu.make_async_copy(buf.at[pl.ds(pl_last * _GC, _GC)],
                              acc.at[didx.at[0]], ssem.at[pl_last]).wait()
        plsc.subcore_barrier()
        pltpu.sync_copy(acc.at[pl.ds(s * _SPT, _SPT)],
                        out_hbm.at[pl.ds(s * _SPT, _SPT)])


@functools.lru_cache(maxsize=None)
def _segsum():
    return pl.kernel(
        _segsum_body,
        out_type=jax.ShapeDtypeStruct((_NP, _D), jnp.float32),
        mesh=_mesh(),
        scratch_types=[
            pltpu.VMEM((2 * _ICH, _GC), jnp.int32),
            pltpu.VMEM((2 * _ICH, _GC), jnp.int32),
            pltpu.VMEM((_GD * _GC, _D), jnp.float32),
            pltpu.VMEM_SHARED((_NP, _D), jnp.float32),
            pltpu.SemaphoreType.DMA((_GD,)),
            pltpu.SemaphoreType.DMA((_GD,)),
            pltpu.SemaphoreType.DMA((2,)),
        ],
    )


def _degree_body(dstr, ones_hbm, zeros_hbm, out_hbm, didx, ones_v, acc):
    c = lax.axis_index("c")
    s = lax.axis_index("s")
    w = c * _NS + s
    pltpu.sync_copy(zeros_hbm.at[pl.ds(s * _SPT, _SPT)],
                    acc.at[pl.ds(s * _SPT, _SPT)])
    pltpu.sync_copy(dstr.at[pl.ds(w * _RPW, _RPW)], didx)
    pltpu.sync_copy(ones_hbm, ones_v)
    plsc.subcore_barrier()

    def body(j, carry):
        pltpu.sync_copy(ones_v, acc.at[didx.at[j]], add=True)
        return carry

    lax.fori_loop(0, _RPW, body, 0, unroll=False)
    plsc.subcore_barrier()
    pltpu.sync_copy(acc.at[pl.ds(s * _SPT, _SPT)],
                    out_hbm.at[c].at[pl.ds(s * _SPT, _SPT)])


@functools.lru_cache(maxsize=None)
def _degree():
    return pl.kernel(
        _degree_body,
        out_type=jax.ShapeDtypeStruct((_NC, _NP, _D), jnp.float32),
        mesh=_mesh(),
        scratch_types=[
            pltpu.VMEM((_RPW, _C), jnp.int32),
            pltpu.VMEM((_C, _D), jnp.float32),
            pltpu.VMEM_SHARED((_NP, _D), jnp.float32),
        ],
    )


def _enc_kernel(x_ref, w1_ref, b1_ref, w2_ref, b2_ref, o_ref):
    h = lax.dot_general(x_ref[...], w1_ref[...], (((1,), (1,)), ((), ())),
                        preferred_element_type=jnp.float32) + b1_ref[...]
    h = jnp.maximum(h, 0.0)
    o_ref[: _N, :] = lax.dot_general(h, w2_ref[...], (((1,), (1,)), ((), ())),
                                     preferred_element_type=jnp.float32) + b2_ref[...]
    o_ref[_N:, :] = jnp.zeros((_NP - _N, _D), jnp.float32)


def _encode(x, w1, b1, w2, b2):
    return pl.pallas_call(
        _enc_kernel,
        out_shape=jax.ShapeDtypeStruct((_NP, _D), jnp.float32),
    )(x, w1, b1.reshape(1, -1), w2, b2.reshape(1, -1))


def _layer_kernel(p_ref, cp_ref, z_ref, wl_ref, bl_ref, wr_ref, g_ref, b_ref,
                  o_ref, *, bn):
    psum = p_ref[: _N, :]
    cnt = cp_ref[0, : _N, 0:1] + cp_ref[1, : _N, 0:1]
    agg = psum / jnp.maximum(cnt, 1.0)
    h = (lax.dot_general(agg, wl_ref[...], (((1,), (1,)), ((), ())),
                         preferred_element_type=jnp.float32)
         + lax.dot_general(z_ref[: _N, :], wr_ref[...], (((1,), (1,)), ((), ())),
                           preferred_element_type=jnp.float32)
         + bl_ref[...])
    if bn:
        m = jnp.mean(h, axis=0, keepdims=True)
        v = jnp.mean((h - m) * (h - m), axis=0, keepdims=True)
        h = (h - m) * lax.rsqrt(v + 1e-5) * g_ref[...] + b_ref[...]
        h = jnp.maximum(h, 0.0)
    o_ref[: _N, :] = h
    o_ref[_N:, :] = jnp.zeros((_NP - _N, _D), jnp.float32)


def _layer(p, cp, z, wl, bl, wr, g, b, bn):
    return pl.pallas_call(
        functools.partial(_layer_kernel, bn=bn),
        out_shape=jax.ShapeDtypeStruct((_NP, _D), jnp.float32),
    )(p, cp, z, wl, bl.reshape(1, -1), wr, g.reshape(1, -1), b.reshape(1, -1))


def _final_kernel(p_ref, cp_ref, z_ref, wl_ref, bl_ref, wr_ref,
                  dw1_ref, db1_ref, dw2_ref, db2_ref, o_ref):
    psum = p_ref[: _N, :]
    cnt = cp_ref[0, : _N, 0:1] + cp_ref[1, : _N, 0:1]
    agg = psum / jnp.maximum(cnt, 1.0)
    h = (lax.dot_general(agg, wl_ref[...], (((1,), (1,)), ((), ())),
                         preferred_element_type=jnp.float32)
         + lax.dot_general(z_ref[: _N, :], wr_ref[...], (((1,), (1,)), ((), ())),
                           preferred_element_type=jnp.float32)
         + bl_ref[...])
    d1 = jnp.maximum(
        lax.dot_general(h, dw1_ref[...], (((1,), (1,)), ((), ())),
                        preferred_element_type=jnp.float32) + db1_ref[...], 0.0)
    o_ref[...] = lax.dot_general(d1, dw2_ref[...], (((1,), (1,)), ((), ())),
                                 preferred_element_type=jnp.float32) + db2_ref[...]


def _final(p, cp, z, wl, bl, wr, dw1, db1, dw2, db2):
    return pl.pallas_call(
        _final_kernel,
        out_shape=jax.ShapeDtypeStruct((_N, 4), jnp.float32),
    )(p, cp, z, wl, bl.reshape(1, -1), wr,
      dw1, db1.reshape(1, -1), dw2, db2.reshape(1, -1))


def kernel(x, edge_index, params):
    src = edge_index[0]
    dst = edge_index[1]
    pad = _EPAD - _E
    sink = _N + (jnp.arange(pad, dtype=jnp.int32) % (_NP - _N))
    spread = jnp.arange(pad, dtype=jnp.int32) % _N
    src_p = jnp.concatenate([src, jnp.full((pad,), _N, jnp.int32)])
    dstr_d = jnp.concatenate([dst, sink]).reshape(_ROWS, _C)
    srcr_g = src_p.reshape(_GROWS, _GC)
    dstr_g = jnp.concatenate([dst, spread]).reshape(_GROWS, _GC)
    zeros_d = jnp.zeros((_NP, _D), jnp.float32)
    ones_c = jnp.ones((_C, _D), jnp.float32)

    p = params
    cp = _degree()(dstr_d, ones_c, zeros_d)
    z = _encode(x, p["enc_W1"], p["enc_b1"], p["enc_W2"], p["enc_b2"])
    for i in range(3):
        ps = _segsum()(z, srcr_g, dstr_g, zeros_d)
        z = _layer(ps, cp, z, p[f"sage{i}_Wl"], p[f"sage{i}_bl"],
                   p[f"sage{i}_Wr"], p[f"bn{i}_g"], p[f"bn{i}_b"], bn=True)
    ps = _segsum()(z, srcr_g, dstr_g, zeros_d)
    return _final(ps, cp, z, p["sage3_Wl"], p["sage3_bl"], p["sage3_Wr"],
                  p["dec_W1"], p["dec_b1"], p["dec_W2"], p["dec_b2"])


# speedup vs baseline: 1.3509x; 1.3509x over previous
import functools

import jax
import jax.numpy as jnp
from jax import lax
from jax.experimental import pallas as pl
from jax.experimental.pallas import tpu as pltpu
from jax.experimental.pallas import tpu_sc as plsc

_N = 10000
_E = 320000
_D = 128
_NC = 2
_NS = 16
_NW = _NC * _NS
_C = 128
_ROWS = 2560
_EPAD = _ROWS * _C
_RPW = _ROWS // _NW
_GC = 64
_GROWS = _EPAD // _GC
_GD = 5
_ICH = 16
_R0 = 288
_R1 = (_GROWS - _NS * _R0) // _NS
_NP = 10112
_SPT = _NP // _NS


def _mesh():
    return plsc.VectorSubcoreMesh(
        core_axis_name="c", subcore_axis_name="s",
        num_cores=_NC, num_subcores=_NS,
    )


def _segsum_body(z_hbm, srcr, dstr, zeros_hbm, out_hbm,
                 sidx, didx, buf, acc, gsem, ssem, isem):
    c = lax.axis_index("c")
    s = lax.axis_index("s")
    rpw = jnp.where(c == 0, _R0, _R1)
    base = jnp.where(c == 0, s * _R0, _NS * _R0 + s * _R1)
    nch = rpw // _ICH
    pltpu.sync_copy(zeros_hbm.at[pl.ds(s * _SPT, _SPT)],
                    acc.at[pl.ds(s * _SPT, _SPT)])
    pltpu.sync_copy(srcr.at[pl.ds(base, _ICH)], sidx.at[pl.ds(0, _ICH)])
    pltpu.sync_copy(dstr.at[pl.ds(base, _ICH)], didx.at[pl.ds(0, _ICH)])
    pltpu.async_copy(srcr.at[pl.ds(base + _ICH, _ICH)],
                     sidx.at[pl.ds(_ICH, _ICH)], isem.at[1])
    pltpu.async_copy(dstr.at[pl.ds(base + _ICH, _ICH)],
                     didx.at[pl.ds(_ICH, _ICH)], isem.at[1])
    plsc.subcore_barrier()

    def body(r, carry):
        ch = r // _ICH
        rr = lax.rem(r, _ICH)
        slot = lax.rem(ch, 2)

        @pl.when(jnp.logical_and(rr == 0, jnp.logical_and(r > 0, r < rpw)))
        def _():
            for _k in range(2):
                pltpu.make_async_copy(srcr.at[pl.ds(base, _ICH)],
                                      sidx.at[pl.ds(0, _ICH)],
                                      isem.at[slot]).wait()

        @pl.when(jnp.logical_and(rr == _GD,
                                 jnp.logical_and(ch >= 1, ch + 1 < nch)))
        def _():
            nxt = ch + 1
            nslot = lax.rem(nxt, 2)
            pltpu.async_copy(srcr.at[pl.ds(base + nxt * _ICH, _ICH)],
                             sidx.at[pl.ds(nslot * _ICH, _ICH)],
                             isem.at[nslot])
            pltpu.async_copy(dstr.at[pl.ds(base + nxt * _ICH, _ICH)],
                             didx.at[pl.ds(nslot * _ICH, _ICH)],
                             isem.at[nslot])

        pa = lax.rem(r, _GD)
        sla = buf.at[pl.ds(pa * _GC, _GC)]

        @pl.when(r >= _GD)
        def _():
            pltpu.make_async_copy(sla, acc.at[didx.at[0]],
                                  ssem.at[pa]).wait()

        @pl.when(r < rpw)
        def _():
            pltpu.async_copy(z_hbm.at[sidx.at[slot * _ICH + rr]], sla,
                             gsem.at[pa])

        rb = r - _GD + 1
        pb = lax.rem(rb + _GD, _GD)
        slb = buf.at[pl.ds(pb * _GC, _GC)]

        @pl.when(jnp.logical_and(rb >= 0, rb < rpw))
        def _():
            chb = rb // _ICH
            rowb = lax.rem(chb, 2) * _ICH + lax.rem(rb, _ICH)
            pltpu.make_async_copy(z_hbm.at[sidx.at[0]], slb,
                                  gsem.at[pb]).wait()
            pltpu.async_copy(slb, acc.at[didx.at[rowb]], ssem.at[pb],
                             add=True)

        return carry

    lax.fori_loop(0, rpw + _GD - 1, body, 0, unroll=False)
    pl_last = lax.rem(rpw - 1, _GD)
    pltpu.make_async_copy(buf.at[pl.ds(pl_last * _GC, _GC)],
                          acc.at[didx.at[0]], ssem.at[pl_last]).wait()
    plsc.subcore_barrier()
    pltpu.sync_copy(acc.at[pl.ds(s * _SPT, _SPT)],
                    out_hbm.at[c].at[pl.ds(s * _SPT, _SPT)])


@functools.lru_cache(maxsize=None)
def _segsum():
    return pl.kernel(
        _segsum_body,
        out_type=jax.ShapeDtypeStruct((_NC, _NP, _D), jnp.float32),
        mesh=_mesh(),
        scratch_types=[
            pltpu.VMEM((2 * _ICH, _GC), jnp.int32),
            pltpu.VMEM((2 * _ICH, _GC), jnp.int32),
            pltpu.VMEM((_GD * _GC, _D), jnp.float32),
            pltpu.VMEM_SHARED((_NP, _D), jnp.float32),
            pltpu.SemaphoreType.DMA((_GD,)),
            pltpu.SemaphoreType.DMA((_GD,)),
            pltpu.SemaphoreType.DMA((2,)),
        ],
    )


def _degree_body(dstr, ones_hbm, zeros_hbm, out_hbm, didx, ones_v, acc):
    c = lax.axis_index("c")
    s = lax.axis_index("s")
    w = c * _NS + s
    pltpu.sync_copy(zeros_hbm.at[pl.ds(s * _SPT, _SPT)],
                    acc.at[pl.ds(s * _SPT, _SPT)])
    pltpu.sync_copy(dstr.at[pl.ds(w * _RPW, _RPW)], didx)
    pltpu.sync_copy(ones_hbm, ones_v)
    plsc.subcore_barrier()

    def body(j, carry):
        pltpu.sync_copy(ones_v, acc.at[didx.at[j]], add=True)
        return carry

    lax.fori_loop(0, _RPW, body, 0, unroll=False)
    plsc.subcore_barrier()
    pltpu.sync_copy(acc.at[pl.ds(s * _SPT, _SPT)],
                    out_hbm.at[c].at[pl.ds(s * _SPT, _SPT)])


@functools.lru_cache(maxsize=None)
def _degree():
    return pl.kernel(
        _degree_body,
        out_type=jax.ShapeDtypeStruct((_NC, _NP, _D), jnp.float32),
        mesh=_mesh(),
        scratch_types=[
            pltpu.VMEM((_RPW, _C), jnp.int32),
            pltpu.VMEM((_C, _D), jnp.float32),
            pltpu.VMEM_SHARED((_NP, _D), jnp.float32),
        ],
    )


def _enc_kernel(x_ref, w1_ref, b1_ref, w2_ref, b2_ref, o_ref):
    h = lax.dot_general(x_ref[...], w1_ref[...], (((1,), (1,)), ((), ())),
                        preferred_element_type=jnp.float32) + b1_ref[...]
    h = jnp.maximum(h, 0.0)
    o_ref[...] = lax.dot_general(h, w2_ref[...], (((1,), (1,)), ((), ())),
                                 preferred_element_type=jnp.float32) + b2_ref[...]


def _encode(x, w1, b1, w2, b2):
    return pl.pallas_call(
        _enc_kernel,
        out_shape=jax.ShapeDtypeStruct((_N, _D), jnp.float32),
    )(x, w1, b1.reshape(1, -1), w2, b2.reshape(1, -1))


def _layer_kernel(p_ref, cp_ref, z_ref, wl_ref, bl_ref, wr_ref, g_ref, b_ref,
                  o_ref, *, bn):
    psum = p_ref[0, : _N, :] + p_ref[1, : _N, :]
    cnt = cp_ref[0, : _N, 0:1] + cp_ref[1, : _N, 0:1]
    agg = psum / jnp.maximum(cnt, 1.0)
    h = (lax.dot_general(agg, wl_ref[...], (((1,), (1,)), ((), ())),
                         preferred_element_type=jnp.float32)
         + lax.dot_general(z_ref[...], wr_ref[...], (((1,), (1,)), ((), ())),
                           preferred_element_type=jnp.float32)
         + bl_ref[...])
    if bn:
        m = jnp.mean(h, axis=0, keepdims=True)
        v = jnp.mean((h - m) * (h - m), axis=0, keepdims=True)
        h = (h - m) * lax.rsqrt(v + 1e-5) * g_ref[...] + b_ref[...]
        h = jnp.maximum(h, 0.0)
    o_ref[...] = h


def _layer(p, cp, z, wl, bl, wr, g, b, bn):
    return pl.pallas_call(
        functools.partial(_layer_kernel, bn=bn),
        out_shape=jax.ShapeDtypeStruct((_N, _D), jnp.float32),
    )(p, cp, z, wl, bl.reshape(1, -1), wr, g.reshape(1, -1), b.reshape(1, -1))


def _final_kernel(p_ref, cp_ref, z_ref, wl_ref, bl_ref, wr_ref,
                  dw1_ref, db1_ref, dw2_ref, db2_ref, o_ref):
    psum = p_ref[0, : _N, :] + p_ref[1, : _N, :]
    cnt = cp_ref[0, : _N, 0:1] + cp_ref[1, : _N, 0:1]
    agg = psum / jnp.maximum(cnt, 1.0)
    h = (lax.dot_general(agg, wl_ref[...], (((1,), (1,)), ((), ())),
                         preferred_element_type=jnp.float32)
         + lax.dot_general(z_ref[...], wr_ref[...], (((1,), (1,)), ((), ())),
                           preferred_element_type=jnp.float32)
         + bl_ref[...])
    d1 = jnp.maximum(
        lax.dot_general(h, dw1_ref[...], (((1,), (1,)), ((), ())),
                        preferred_element_type=jnp.float32) + db1_ref[...], 0.0)
    o_ref[...] = lax.dot_general(d1, dw2_ref[...], (((1,), (1,)), ((), ())),
                                 preferred_element_type=jnp.float32) + db2_ref[...]


def _final(p, cp, z, wl, bl, wr, dw1, db1, dw2, db2):
    return pl.pallas_call(
        _final_kernel,
        out_shape=jax.ShapeDtypeStruct((_N, 4), jnp.float32),
    )(p, cp, z, wl, bl.reshape(1, -1), wr,
      dw1, db1.reshape(1, -1), dw2, db2.reshape(1, -1))


def kernel(x, edge_index, params):
    src = edge_index[0]
    dst = edge_index[1]
    pad = _EPAD - _E
    sink = _N + (jnp.arange(pad, dtype=jnp.int32) % (_NP - _N))
    src_p = jnp.concatenate([src, jnp.zeros((pad,), jnp.int32)])
    dst_p = jnp.concatenate([dst, sink])
    dstr_d = dst_p.reshape(_ROWS, _C)
    srcr_g = src_p.reshape(_GROWS, _GC)
    dstr_g = dst_p.reshape(_GROWS, _GC)
    zeros_d = jnp.zeros((_NP, _D), jnp.float32)
    ones_c = jnp.ones((_C, _D), jnp.float32)

    p = params
    cp = _degree()(dstr_d, ones_c, zeros_d)
    z = _encode(x, p["enc_W1"], p["enc_b1"], p["enc_W2"], p["enc_b2"])
    for i in range(3):
        ps = _segsum()(z, srcr_g, dstr_g, zeros_d)
        z = _layer(ps, cp, z, p[f"sage{i}_Wl"], p[f"sage{i}_bl"],
                   p[f"sage{i}_Wr"], p[f"bn{i}_g"], p[f"bn{i}_b"], bn=True)
    ps = _segsum()(z, srcr_g, dstr_g, zeros_d)
    return _final(ps, cp, z, p["sage3_Wl"], p["sage3_bl"], p["sage3_Wr"],
                  p["dec_W1"], p["dec_b1"], p["dec_W2"], p["dec_b2"])
